# baseline (speedup 1.0000x reference)
import functools
import math

import jax
import jax.numpy as jnp
from jax.experimental import pallas as pl
from jax.experimental.pallas import tpu as pltpu

D_MODEL = 512
D_FF = 1024
N_HEADS = 8
N_EXPERTS = 6
HEAD_DIM = D_MODEL // N_HEADS


def _mm_kernel(x_ref, w_ref, b_ref, o_ref):
    o_ref[...] = (
        jnp.dot(x_ref[...], w_ref[...], preferred_element_type=jnp.float32)
        + b_ref[...]
    )


def _mm(x, w, b):
    n, k = x.shape
    m = w.shape[1]
    return pl.pallas_call(
        _mm_kernel,
        out_shape=jax.ShapeDtypeStruct((n, m), jnp.float32),
    )(x, w, b.reshape(1, m))


def _ln_op(y, g, b):
    mu = jnp.mean(y, axis=-1, keepdims=True)
    var = jnp.mean((y - mu) ** 2, axis=-1, keepdims=True)
    return (y - mu) * jax.lax.rsqrt(var + 1e-5) * g + b


def _mm_res_ln_kernel(x_ref, w_ref, b_ref, res_ref, g_ref, gb_ref, o_ref):
    y = (
        jnp.dot(x_ref[...], w_ref[...], preferred_element_type=jnp.float32)
        + b_ref[...]
    )
    y = y + res_ref[...]
    o_ref[...] = _ln_op(y, g_ref[...], gb_ref[...])


def _mm_res_ln(x, w, b, res, g, gb):
    n, k = x.shape
    m = w.shape[1]
    return pl.pallas_call(
        _mm_res_ln_kernel,
        out_shape=jax.ShapeDtypeStruct((n, m), jnp.float32),
    )(x, w, b.reshape(1, m), res, g.reshape(1, m), gb.reshape(1, m))


def _ln_kernel(x_ref, g_ref, b_ref, o_ref):
    o_ref[...] = _ln_op(x_ref[...], g_ref[...], b_ref[...])


def _ln(x, g, b):
    n, d = x.shape
    return pl.pallas_call(
        _ln_kernel,
        out_shape=jax.ShapeDtypeStruct((n, d), jnp.float32),
    )(x, g.reshape(1, d), b.reshape(1, d))


def _attn_kernel(q_ref, k_ref, v_ref, m_ref, o_ref):
    q = q_ref[0, 0]
    k = k_ref[0, 0]
    v = v_ref[0, 0]
    s = jax.lax.dot_general(
        q, k, (((1,), (1,)), ((), ())), preferred_element_type=jnp.float32
    ) * (1.0 / math.sqrt(HEAD_DIM))
    s = s + m_ref[...]
    mx = jnp.max(s, axis=-1, keepdims=True)
    e = jnp.exp(s - mx)
    a = e / jnp.sum(e, axis=-1, keepdims=True)
    o_ref[0, 0] = jnp.dot(a, v, preferred_element_type=jnp.float32)


def _attn_core(qh, kh, vh, mask):
    bb, h, tq, hd = qh.shape
    tk = kh.shape[2]
    return pl.pallas_call(
        _attn_kernel,
        grid=(bb, h),
        in_specs=[
            pl.BlockSpec((1, 1, tq, hd), lambda i, j: (i, j, 0, 0)),
            pl.BlockSpec((1, 1, tk, hd), lambda i, j: (i, j, 0, 0)),
            pl.BlockSpec((1, 1, tk, hd), lambda i, j: (i, j, 0, 0)),
            pl.BlockSpec((tq, tk), lambda i, j: (0, 0)),
        ],
        out_specs=pl.BlockSpec((1, 1, tq, hd), lambda i, j: (i, j, 0, 0)),
        out_shape=jax.ShapeDtypeStruct((bb, h, tq, hd), jnp.float32),
    )(qh, kh, vh, mask)


def _split_heads(y, t, b):
    return y.reshape(t, b, N_HEADS, HEAD_DIM).transpose(1, 2, 0, 3)


def _merge_heads(o, t, b):
    return o.transpose(2, 0, 1, 3).reshape(t * b, D_MODEL)


def _mha(xq, xkv, p, mask, tq, tk, b):
    wq, wk, wv = jnp.split(p["Wqkv"], 3, axis=1)
    bq, bk, bv = jnp.split(p["bqkv"], 3)
    if xq is xkv:
        qkv = _mm(xq, p["Wqkv"], p["bqkv"])
        q, k, v = jnp.split(qkv, 3, axis=1)
    else:
        q = _mm(xq, wq, bq)
        kv = _mm(xkv, jnp.concatenate([wk, wv], axis=1), jnp.concatenate([bk, bv]))
        k, v = jnp.split(kv, 2, axis=1)
    qh = _split_heads(q, tq, b)
    kh = _split_heads(k, tk, b)
    vh = _split_heads(v, tk, b)
    o = _attn_core(qh, kh, vh, mask)
    return _merge_heads(o, tq, b)


def _moe_kernel(x_ref, wr_ref, wg_ref, bg_ref, wu_ref, bu_ref, wd_ref, bd_ref,
                g_ref, gb_ref, o_ref, w_scratch, acc):
    e = pl.program_id(0)

    @pl.when(e == 0)
    def _router():
        logits = jnp.dot(x_ref[...], wr_ref[...], preferred_element_type=jnp.float32)
        mx = jnp.max(logits, axis=-1, keepdims=True)
        ex = jnp.exp(logits - mx)
        p = ex / jnp.sum(ex, axis=-1, keepdims=True)
        ids = jax.lax.broadcasted_iota(jnp.int32, p.shape, 1)
        i1 = jnp.argmax(p, axis=-1, keepdims=True)
        m1 = jnp.max(p, axis=-1, keepdims=True)
        p2 = jnp.where(ids == i1, -jnp.inf, p)
        i2 = jnp.argmax(p2, axis=-1, keepdims=True)
        m2 = jnp.max(p2, axis=-1, keepdims=True)
        denom = m1 + m2
        w = jnp.where(ids == i1, m1, 0.0) + jnp.where(ids == i2, m2, 0.0)
        w_scratch[...] = w / denom

    x = x_ref[...]
    h = jnp.dot(x, wg_ref[0], preferred_element_type=jnp.float32) + bg_ref[0]
    u = jnp.dot(x, wu_ref[0], preferred_element_type=jnp.float32) + bu_ref[0]
    act = jax.nn.silu(h) * u
    y = jnp.dot(act, wd_ref[0], preferred_element_type=jnp.float32) + bd_ref[0]
    wv = w_scratch[...]
    lane = jax.lax.broadcasted_iota(jnp.int32, wv.shape, 1)
    wcol = jnp.sum(jnp.where(lane == e, wv, 0.0), axis=1, keepdims=True)
    contrib = y * wcol

    @pl.when(e == 0)
    def _init():
        acc[...] = contrib

    @pl.when(e > 0)
    def _accum():
        acc[...] = acc[...] + contrib

    @pl.when(e == N_EXPERTS - 1)
    def _finish():
        o_ref[...] = _ln_op(x + acc[...], g_ref[...], gb_ref[...])


def _moe_ln(x, p, g, gb):
    n, d = x.shape
    return pl.pallas_call(
        _moe_kernel,
        grid=(N_EXPERTS,),
        in_specs=[
            pl.BlockSpec((n, d), lambda e: (0, 0)),
            pl.BlockSpec((d, N_EXPERTS), lambda e: (0, 0)),
            pl.BlockSpec((1, d, D_FF), lambda e: (e, 0, 0)),
            pl.BlockSpec((1, 1, D_FF), lambda e: (e, 0, 0)),
            pl.BlockSpec((1, d, D_FF), lambda e: (e, 0, 0)),
            pl.BlockSpec((1, 1, D_FF), lambda e: (e, 0, 0)),
            pl.BlockSpec((1, D_FF, d), lambda e: (e, 0, 0)),
            pl.BlockSpec((1, 1, d), lambda e: (e, 0, 0)),
            pl.BlockSpec((1, d), lambda e: (0, 0)),
            pl.BlockSpec((1, d), lambda e: (0, 0)),
        ],
        out_specs=pl.BlockSpec((n, d), lambda e: (0, 0)),
        out_shape=jax.ShapeDtypeStruct((n, d), jnp.float32),
        scratch_shapes=[
            pltpu.VMEM((n, N_EXPERTS), jnp.float32),
            pltpu.VMEM((n, d), jnp.float32),
        ],
    )(x, p["Wr"], p["Wg"], p["bg"][:, None, :], p["Wu"], p["bu"][:, None, :],
      p["Wd"], p["bd"][:, None, :], g.reshape(1, d), gb.reshape(1, d))


def kernel(x, x_root, x_attr, feature_semantic, feature_key, feature_scene_offset,
           feature_motion, feature_emotion, params):
    bsz, t_chord = x_root.shape
    t_video = feature_scene_offset.shape[1]

    xr = jnp.take(params["emb_root"], x_root, axis=0)
    xa = jnp.take(params["emb_attr"], x_attr, axis=0)
    xe = xr + xa
    fkey = jnp.broadcast_to(feature_key[:, 0][:, None, None], (bsz, t_chord, 1))
    xc = jnp.concatenate([xe, fkey], axis=-1)
    xf = _mm(xc.reshape(bsz * t_chord, D_MODEL + 1), params["Wc"], params["bc"])
    xf = xf.reshape(bsz, t_chord, D_MODEL)

    vf_concat = jnp.concatenate([
        feature_semantic.astype(jnp.float32),
        feature_scene_offset[..., None].astype(jnp.float32),
        feature_motion[..., None].astype(jnp.float32),
        feature_emotion.astype(jnp.float32)], axis=-1)
    vdim = vf_concat.shape[-1]
    vf = _mm(vf_concat.reshape(bsz * t_video, vdim), params["Wv"], params["bv"])
    vf = vf.reshape(bsz, t_video, D_MODEL)

    xf = xf.transpose(1, 0, 2) + params["pos"][:t_chord, None, :]
    vf = vf.transpose(1, 0, 2) + params["pos_video"][:t_video, None, :]
    xf = xf.reshape(t_chord * bsz, D_MODEL)
    vf = vf.reshape(t_video * bsz, D_MODEL)

    zero_mask = jnp.zeros((t_video, t_video), jnp.float32)
    causal_mask = jnp.where(
        jnp.tril(jnp.ones((t_chord, t_chord), dtype=bool)), 0.0, -jnp.inf
    ).astype(jnp.float32)

    h = vf
    for p in params["enc_layers"]:
        a = _mha(h, h, p["attn"], zero_mask, t_video, t_video, bsz)
        h = _mm_res_ln(a, p["attn"]["Wo"], p["attn"]["bo"], h,
                       p["ln1"]["g"], p["ln1"]["b"])
        h = _moe_ln(h, p["moe"], p["ln2"]["g"], p["ln2"]["b"])
    mem = _ln(h, params["enc_norm"]["g"], params["enc_norm"]["b"])

    h = xf
    for p in params["dec_layers"]:
        a = _mha(h, h, p["sattn"], causal_mask, t_chord, t_chord, bsz)
        h = _mm_res_ln(a, p["sattn"]["Wo"], p["sattn"]["bo"], h,
                       p["ln1"]["g"], p["ln1"]["b"])
        a = _mha(h, mem, p["xattn"], zero_mask, t_chord, t_video, bsz)
        h = _mm_res_ln(a, p["xattn"]["Wo"], p["xattn"]["bo"], h,
                       p["ln2"]["g"], p["ln2"]["b"])
        h = _moe_ln(h, p["moe"], p["ln3"]["g"], p["ln3"]["b"])
    out = _ln(h, params["dec_norm"]["g"], params["dec_norm"]["b"])

    out = out.reshape(t_chord, bsz, D_MODEL).transpose(1, 0, 2)
    y = _mm(out.reshape(bsz * t_chord, D_MODEL), params["Wout"], params["bout"])
    return y.reshape(bsz, t_chord, -1)
